# baseline (speedup 1.0000x reference)
import functools

import jax
import jax.numpy as jnp
from jax import lax
from jax.experimental import pallas as pl
from jax.experimental.pallas import tpu as pltpu
from jax.experimental.pallas import tpu_sc as plsc

N = 10000
E = 320000
D = 128

R = 10240
BLK = 512
GRID = R // BLK
CHUNK = 128
NWORK = 32
NBUF = 2
C0 = 56
C1 = 104
CMAX = max(C0, C1)
TCHUNK = 16 * (C0 + C1)
EP = TCHUNK * CHUNK
TPAD = max(TCHUNK, 16 * C0 + 15 * C1 + CMAX)
RPT = R // 16


def _elu(v):
    return jnp.where(v > 0, v, jnp.exp(v) - 1.0)


def _tc_a_body(x_ref, w1_ref, al_ref, ar_ref, out_ref, er_ref):
    feat = jnp.dot(x_ref[...], w1_ref[...], preferred_element_type=jnp.float32)
    el = jnp.dot(feat, al_ref[...], preferred_element_type=jnp.float32)
    er = jnp.dot(feat, ar_ref[...], preferred_element_type=jnp.float32)
    z8 = jnp.zeros((BLK, 8), jnp.float32)
    out_ref[...] = jnp.concatenate([feat, el, z8], axis=1)
    er_ref[...] = jnp.concatenate([er, z8], axis=1)


def _tc_a(xp, W1, A_l, A_r):
    return pl.pallas_call(
        _tc_a_body,
        grid=(GRID,),
        in_specs=[
            pl.BlockSpec((BLK, D), lambda i: (i, 0)),
            pl.BlockSpec((D, 64), lambda i: (0, 0)),
            pl.BlockSpec((64, 8), lambda i: (0, 0)),
            pl.BlockSpec((64, 8), lambda i: (0, 0)),
        ],
        out_specs=[
            pl.BlockSpec((BLK, 80), lambda i: (i, 0)),
            pl.BlockSpec((BLK, 16), lambda i: (i, 0)),
        ],
        out_shape=[
            jax.ShapeDtypeStruct((R, 80), jnp.float32),
            jax.ShapeDtypeStruct((R, 16), jnp.float32),
        ],
    )(xp, W1, A_l, A_r)


def _tc_b_body(acc_ref, w2_ref, p_ref, b1_ref, a2l_ref, a2r_ref,
               out_ref, er_ref):
    t = acc_ref[0] + acc_ref[1]
    s = t[:, 64:72]
    sinv = 1.0 / jnp.where(s > 0, s, 1.0)
    sexp = jnp.dot(sinv, p_ref[...], preferred_element_type=jnp.float32)
    h = _elu(_elu(t[:, :64] * sexp + b1_ref[...]))
    feat2 = jnp.dot(h, w2_ref[...], preferred_element_type=jnp.float32)
    el2 = jnp.dot(feat2, a2l_ref[...], preferred_element_type=jnp.float32)
    er2 = jnp.dot(feat2, a2r_ref[...], preferred_element_type=jnp.float32)
    z24 = jnp.zeros((BLK, 24), jnp.float32)
    out_ref[...] = jnp.concatenate([feat2, z24, el2], axis=1)
    er_ref[...] = er2


def _tc_b(acc1, W2, P, b1, A2l, A2r):
    return pl.pallas_call(
        _tc_b_body,
        grid=(GRID,),
        in_specs=[
            pl.BlockSpec((2, BLK, 80), lambda i: (0, i, 0)),
            pl.BlockSpec((64, 40), lambda i: (0, 0)),
            pl.BlockSpec((8, 64), lambda i: (0, 0)),
            pl.BlockSpec((1, 64), lambda i: (0, 0)),
            pl.BlockSpec((40, 16), lambda i: (0, 0)),
            pl.BlockSpec((40, 16), lambda i: (0, 0)),
        ],
        out_specs=[
            pl.BlockSpec((BLK, 80), lambda i: (i, 0)),
            pl.BlockSpec((BLK, 16), lambda i: (i, 0)),
        ],
        out_shape=[
            jax.ShapeDtypeStruct((R, 80), jnp.float32),
            jax.ShapeDtypeStruct((R, 16), jnp.float32),
        ],
    )(acc1, W2, P, b1, A2l, A2r)


def _tc_c_body(acc_ref, b2_ref, out_ref):
    t = acc_ref[0] + acc_ref[1]
    s = t[:, 64:65]
    sinv = 1.0 / jnp.where(s > 0, s, 1.0)
    out_ref[...] = t[:, :40] * sinv + b2_ref[...]


def _tc_c(acc2, b2):
    return pl.pallas_call(
        _tc_c_body,
        grid=(GRID,),
        in_specs=[
            pl.BlockSpec((2, BLK, 80), lambda i: (0, i, 0)),
            pl.BlockSpec((1, 40), lambda i: (0, 0)),
        ],
        out_specs=pl.BlockSpec((BLK, 40), lambda i: (i, 0)),
        out_shape=jax.ShapeDtypeStruct((N, 40), jnp.float32),
    )(acc2, b2)


def _bcast_heads(wv, k):
    col = (jnp.arange(16, dtype=jnp.int32) >> 3) + 2 * k
    dnums = lax.GatherDimensionNumbers(
        offset_dims=(), collapsed_slice_dims=(0,), start_index_map=(0,))
    return lax.gather(wv, col[:, None], dnums, slice_sizes=(1,),
                      mode=lax.GatherScatterMode.PROMISE_IN_BOUNDS)


def _make_sc_edge(width, el_off, per_head):
    nfv = el_off // 16

    mesh = plsc.VectorSubcoreMesh(core_axis_name="c", subcore_axis_name="s",
                                  num_cores=2, num_subcores=16)

    scratch = (
        [pltpu.VMEM((CMAX, CHUNK), jnp.int32) for _ in range(2)]
        + [pltpu.VMEM((CHUNK, width), jnp.float32) for _ in range(NBUF)]
        + [pltpu.VMEM((CHUNK, 16), jnp.float32) for _ in range(NBUF)]
        + [pltpu.VMEM((CHUNK, width), jnp.float32) for _ in range(NBUF)]
        + [pltpu.VMEM_SHARED((R, width), jnp.float32)]
        + [pltpu.SemaphoreType.DMA for _ in range(2 * NBUF)]
    )

    @functools.partial(
        pl.kernel,
        out_type=jax.ShapeDtypeStruct((2, R, width), jnp.float32),
        mesh=mesh,
        compiler_params=pltpu.CompilerParams(use_tc_tiling_on_sc=False,
                                             needs_layout_passes=False),
        scratch_types=scratch,
    )
    def edge_kernel(feat_hbm, er_hbm, src_hbm, dst_hbm, out_hbm, *refs):
        idx_src, idx_dst = refs[0:2]
        rows = refs[2:2 + NBUF]
        errows = refs[2 + NBUF:2 + 2 * NBUF]
        scaled = refs[2 + 2 * NBUF:2 + 3 * NBUF]
        acc = refs[2 + 3 * NBUF]
        gsem = refs[3 + 3 * NBUF:3 + 4 * NBUF]
        ssem = refs[3 + 4 * NBUF:3 + 5 * NBUF]

        cid = lax.axis_index("c")
        sid = lax.axis_index("s")
        cw = C0 + cid * (C1 - C0)
        start = cid * (16 * C0) + sid * cw

        def fire_gather(ci, b):
            pltpu.async_copy(feat_hbm.at[idx_src.at[ci]], rows[b], gsem[b])
            pltpu.async_copy(er_hbm.at[idx_dst.at[ci]], errows[b], gsem[b])

        def wait_gather(ci, b):
            pltpu.make_async_copy(feat_hbm.at[idx_src.at[ci]], rows[b],
                                  gsem[b]).wait()
            pltpu.make_async_copy(er_hbm.at[idx_dst.at[ci]], errows[b],
                                  gsem[b]).wait()

        def fire_scatter(ci, b):
            pltpu.async_copy(scaled[b], acc.at[idx_dst.at[ci]], ssem[b],
                             add=True)

        def wait_scatter(ci, b):
            pltpu.make_async_copy(scaled[b], acc.at[idx_dst.at[ci]],
                                  ssem[b]).wait()

        def compute(b):
            def edge_body(i, carry):
                for u in range(4):
                    e = i * 4 + u
                    vel = rows[b][e, pl.ds(el_off, 16)]
                    ver = errows[b][e, pl.ds(0, 16)]
                    t = vel + ver
                    wv = jnp.exp(jnp.maximum(t, 0.2 * t))
                    scaled[b][e, pl.ds(el_off, 16)] = wv
                    for k in range(nfv):
                        wk = _bcast_heads(wv, k) if per_head else wv
                        scaled[b][e, pl.ds(16 * k, 16)] = (
                            rows[b][e, pl.ds(16 * k, 16)] * wk)
                return carry
            lax.fori_loop(0, CHUNK // 4, edge_body, 0)

        pltpu.sync_copy(src_hbm.at[pl.ds(start, CMAX)], idx_src)
        pltpu.sync_copy(dst_hbm.at[pl.ds(start, CMAX)], idx_dst)

        def zrow(i, carry):
            for k in range(width // 16):
                scaled[0][i, pl.ds(16 * k, 16)] = jnp.zeros((16,), jnp.float32)
            return carry
        lax.fori_loop(0, CHUNK, zrow, 0)
        for j in range(RPT // CHUNK):
            pltpu.sync_copy(scaled[0],
                            acc.at[pl.ds(sid * RPT + j * CHUNK, CHUNK)])

        for b in range(NBUF):
            fire_gather(b, b)
        plsc.subcore_barrier()

        def group_body(g, carry):
            for b in range(NBUF):
                ci = g * NBUF + b
                wait_gather(ci, b)

                @pl.when(g >= 1)
                def _drain():
                    wait_scatter(ci - NBUF, b)
                compute(b)
                fire_scatter(ci, b)

                @pl.when(ci + NBUF < cw)
                def _prefetch():
                    fire_gather(ci + NBUF, b)
            return carry
        lax.fori_loop(0, cw // NBUF, group_body, 0)
        for b in range(NBUF):
            wait_scatter(cw - NBUF + b, b)
        plsc.subcore_barrier()

        pltpu.sync_copy(acc.at[pl.ds(sid * RPT, RPT)],
                        out_hbm.at[cid, pl.ds(sid * RPT, RPT)])

    return edge_kernel


@functools.lru_cache(maxsize=None)
def _sc_edge(width, el_off, per_head):
    return _make_sc_edge(width=width, el_off=el_off, per_head=per_head)


def kernel(x, edge_index, W1, attn_l1, attn_r1, b1, W2, attn_l2, attn_r2, b2):
    f32 = jnp.float32
    xp = jnp.zeros((R, D), f32).at[:N].set(x)

    eye8 = jnp.eye(8, dtype=f32)
    A_l = (attn_l1[:, :, None] * eye8[:, None, :]).reshape(64, 8)
    A_r = (attn_r1[:, :, None] * eye8[:, None, :]).reshape(64, 8)
    P = jnp.repeat(eye8, 8, axis=1)
    A2l = jnp.broadcast_to(attn_l2.reshape(40, 1), (40, 16)).astype(f32)
    A2r = jnp.broadcast_to(attn_r2.reshape(40, 1), (40, 16)).astype(f32)

    pad = EP - E
    tailpad = (TPAD - TCHUNK) * CHUNK
    srcp = jnp.concatenate([edge_index[0], jnp.zeros((pad + tailpad,),
                                                     jnp.int32)])
    dstp = jnp.concatenate([edge_index[1], jnp.full((pad,), N, jnp.int32),
                            jnp.zeros((tailpad,), jnp.int32)])
    srcp = srcp.reshape(TPAD, CHUNK)
    dstp = dstp.reshape(TPAD, CHUNK)

    table1, er1 = _tc_a(xp, W1, A_l, A_r)
    acc1 = _sc_edge(80, 64, True)(table1, er1, srcp, dstp)
    table2, er2 = _tc_b(acc1, W2, P, b1.reshape(1, 64), A2l, A2r)
    acc2 = _sc_edge(80, 64, True)(table2, er2, srcp, dstp)
    return _tc_c(acc2, b2.reshape(1, 40))


# speedup vs baseline: 1.1876x; 1.1876x over previous
import functools

import jax
import jax.numpy as jnp
from jax import lax
from jax.experimental import pallas as pl
from jax.experimental.pallas import tpu as pltpu
from jax.experimental.pallas import tpu_sc as plsc

N = 10000
E = 320000
D = 128

R = 10240
BLK = 512
GRID = R // BLK
CHUNK = 128
NWORK = 32
NBUF = 2
C0 = 104
C1 = 56
CMAX = max(C0, C1)
TCHUNK = 16 * (C0 + C1)
EP = TCHUNK * CHUNK
TPAD = max(TCHUNK, 16 * C0 + 15 * C1 + CMAX)
RPT = R // 16


def _elu(v):
    return jnp.where(v > 0, v, jnp.exp(v) - 1.0)


def _tc_a_body(x_ref, w1_ref, al_ref, ar_ref, out_ref, er_ref):
    feat = jnp.dot(x_ref[...], w1_ref[...], preferred_element_type=jnp.float32)
    el = jnp.dot(feat, al_ref[...], preferred_element_type=jnp.float32)
    er = jnp.dot(feat, ar_ref[...], preferred_element_type=jnp.float32)
    z8 = jnp.zeros((BLK, 8), jnp.float32)
    out_ref[...] = jnp.concatenate([feat, el, z8], axis=1)
    er_ref[...] = jnp.concatenate([er, z8], axis=1)


def _tc_a(xp, W1, A_l, A_r):
    return pl.pallas_call(
        _tc_a_body,
        grid=(GRID,),
        in_specs=[
            pl.BlockSpec((BLK, D), lambda i: (i, 0)),
            pl.BlockSpec((D, 64), lambda i: (0, 0)),
            pl.BlockSpec((64, 8), lambda i: (0, 0)),
            pl.BlockSpec((64, 8), lambda i: (0, 0)),
        ],
        out_specs=[
            pl.BlockSpec((BLK, 80), lambda i: (i, 0)),
            pl.BlockSpec((BLK, 16), lambda i: (i, 0)),
        ],
        out_shape=[
            jax.ShapeDtypeStruct((R, 80), jnp.float32),
            jax.ShapeDtypeStruct((R, 16), jnp.float32),
        ],
    )(xp, W1, A_l, A_r)


def _tc_b_body(acc_ref, w2_ref, p_ref, b1_ref, a2l_ref, a2r_ref,
               out_ref, er_ref):
    t = acc_ref[0] + acc_ref[1]
    s = t[:, 64:72]
    sinv = 1.0 / jnp.where(s > 0, s, 1.0)
    sexp = jnp.dot(sinv, p_ref[...], preferred_element_type=jnp.float32)
    h = _elu(_elu(t[:, :64] * sexp + b1_ref[...]))
    feat2 = jnp.dot(h, w2_ref[...], preferred_element_type=jnp.float32)
    el2 = jnp.dot(feat2, a2l_ref[...], preferred_element_type=jnp.float32)
    er2 = jnp.dot(feat2, a2r_ref[...], preferred_element_type=jnp.float32)
    z24 = jnp.zeros((BLK, 24), jnp.float32)
    out_ref[...] = jnp.concatenate([feat2, z24, el2], axis=1)
    er_ref[...] = er2


def _tc_b(acc1, W2, P, b1, A2l, A2r):
    return pl.pallas_call(
        _tc_b_body,
        grid=(GRID,),
        in_specs=[
            pl.BlockSpec((2, BLK, 80), lambda i: (0, i, 0)),
            pl.BlockSpec((64, 40), lambda i: (0, 0)),
            pl.BlockSpec((8, 64), lambda i: (0, 0)),
            pl.BlockSpec((1, 64), lambda i: (0, 0)),
            pl.BlockSpec((40, 16), lambda i: (0, 0)),
            pl.BlockSpec((40, 16), lambda i: (0, 0)),
        ],
        out_specs=[
            pl.BlockSpec((BLK, 80), lambda i: (i, 0)),
            pl.BlockSpec((BLK, 16), lambda i: (i, 0)),
        ],
        out_shape=[
            jax.ShapeDtypeStruct((R, 80), jnp.float32),
            jax.ShapeDtypeStruct((R, 16), jnp.float32),
        ],
    )(acc1, W2, P, b1, A2l, A2r)


def _tc_c_body(acc_ref, b2_ref, out_ref):
    t = acc_ref[0] + acc_ref[1]
    s = t[:, 64:65]
    sinv = 1.0 / jnp.where(s > 0, s, 1.0)
    out_ref[...] = t[:, :40] * sinv + b2_ref[...]


def _tc_c(acc2, b2):
    return pl.pallas_call(
        _tc_c_body,
        grid=(GRID,),
        in_specs=[
            pl.BlockSpec((2, BLK, 80), lambda i: (0, i, 0)),
            pl.BlockSpec((1, 40), lambda i: (0, 0)),
        ],
        out_specs=pl.BlockSpec((BLK, 40), lambda i: (i, 0)),
        out_shape=jax.ShapeDtypeStruct((N, 40), jnp.float32),
    )(acc2, b2)


def _bcast_heads(wv, k):
    col = (jnp.arange(16, dtype=jnp.int32) >> 3) + 2 * k
    dnums = lax.GatherDimensionNumbers(
        offset_dims=(), collapsed_slice_dims=(0,), start_index_map=(0,))
    return lax.gather(wv, col[:, None], dnums, slice_sizes=(1,),
                      mode=lax.GatherScatterMode.PROMISE_IN_BOUNDS)


def _make_sc_edge(width, el_off, per_head):
    nfv = el_off // 16

    mesh = plsc.VectorSubcoreMesh(core_axis_name="c", subcore_axis_name="s",
                                  num_cores=2, num_subcores=16)

    scratch = (
        [pltpu.VMEM((CMAX, CHUNK), jnp.int32) for _ in range(2)]
        + [pltpu.VMEM((CHUNK, width), jnp.float32) for _ in range(NBUF)]
        + [pltpu.VMEM((CHUNK, 16), jnp.float32) for _ in range(NBUF)]
        + [pltpu.VMEM((CHUNK, width), jnp.float32) for _ in range(NBUF)]
        + [pltpu.VMEM_SHARED((R, width), jnp.float32)]
        + [pltpu.SemaphoreType.DMA for _ in range(2 * NBUF)]
    )

    @functools.partial(
        pl.kernel,
        out_type=jax.ShapeDtypeStruct((2, R, width), jnp.float32),
        mesh=mesh,
        compiler_params=pltpu.CompilerParams(use_tc_tiling_on_sc=False,
                                             needs_layout_passes=False),
        scratch_types=scratch,
    )
    def edge_kernel(feat_hbm, er_hbm, src_hbm, dst_hbm, out_hbm, *refs):
        idx_src, idx_dst = refs[0:2]
        rows = refs[2:2 + NBUF]
        errows = refs[2 + NBUF:2 + 2 * NBUF]
        scaled = refs[2 + 2 * NBUF:2 + 3 * NBUF]
        acc = refs[2 + 3 * NBUF]
        gsem = refs[3 + 3 * NBUF:3 + 4 * NBUF]
        ssem = refs[3 + 4 * NBUF:3 + 5 * NBUF]

        cid = lax.axis_index("c")
        sid = lax.axis_index("s")
        cw = C0 + cid * (C1 - C0)
        start = cid * (16 * C0) + sid * cw

        def fire_gather(ci, b):
            pltpu.async_copy(feat_hbm.at[idx_src.at[ci]], rows[b], gsem[b])
            pltpu.async_copy(er_hbm.at[idx_dst.at[ci]], errows[b], gsem[b])

        def wait_gather(ci, b):
            pltpu.make_async_copy(feat_hbm.at[idx_src.at[ci]], rows[b],
                                  gsem[b]).wait()
            pltpu.make_async_copy(er_hbm.at[idx_dst.at[ci]], errows[b],
                                  gsem[b]).wait()

        def fire_scatter(ci, b):
            pltpu.async_copy(scaled[b], acc.at[idx_dst.at[ci]], ssem[b],
                             add=True)

        def wait_scatter(ci, b):
            pltpu.make_async_copy(scaled[b], acc.at[idx_dst.at[ci]],
                                  ssem[b]).wait()

        def compute(b):
            def edge_body(i, carry):
                for u in range(4):
                    e = i * 4 + u
                    vel = rows[b][e, pl.ds(el_off, 16)]
                    ver = errows[b][e, pl.ds(0, 16)]
                    t = vel + ver
                    wv = jnp.exp(jnp.maximum(t, 0.2 * t))
                    scaled[b][e, pl.ds(el_off, 16)] = wv
                    for k in range(nfv):
                        wk = _bcast_heads(wv, k) if per_head else wv
                        scaled[b][e, pl.ds(16 * k, 16)] = (
                            rows[b][e, pl.ds(16 * k, 16)] * wk)
                return carry
            lax.fori_loop(0, CHUNK // 4, edge_body, 0)

        pltpu.sync_copy(src_hbm.at[pl.ds(start, CMAX)], idx_src)
        pltpu.sync_copy(dst_hbm.at[pl.ds(start, CMAX)], idx_dst)

        def zrow(i, carry):
            for k in range(width // 16):
                scaled[0][i, pl.ds(16 * k, 16)] = jnp.zeros((16,), jnp.float32)
            return carry
        lax.fori_loop(0, CHUNK, zrow, 0)
        for j in range(RPT // CHUNK):
            pltpu.sync_copy(scaled[0],
                            acc.at[pl.ds(sid * RPT + j * CHUNK, CHUNK)])

        for b in range(NBUF):
            fire_gather(b, b)
        plsc.subcore_barrier()

        def group_body(g, carry):
            for b in range(NBUF):
                ci = g * NBUF + b
                wait_gather(ci, b)

                @pl.when(g >= 1)
                def _drain():
                    wait_scatter(ci - NBUF, b)
                compute(b)
                fire_scatter(ci, b)

                @pl.when(ci + NBUF < cw)
                def _prefetch():
                    fire_gather(ci + NBUF, b)
            return carry
        lax.fori_loop(0, cw // NBUF, group_body, 0)
        for b in range(NBUF):
            wait_scatter(cw - NBUF + b, b)
        plsc.subcore_barrier()

        pltpu.sync_copy(acc.at[pl.ds(sid * RPT, RPT)],
                        out_hbm.at[cid, pl.ds(sid * RPT, RPT)])

    return edge_kernel


@functools.lru_cache(maxsize=None)
def _sc_edge(width, el_off, per_head):
    return _make_sc_edge(width=width, el_off=el_off, per_head=per_head)


def kernel(x, edge_index, W1, attn_l1, attn_r1, b1, W2, attn_l2, attn_r2, b2):
    f32 = jnp.float32
    xp = jnp.zeros((R, D), f32).at[:N].set(x)

    eye8 = jnp.eye(8, dtype=f32)
    A_l = (attn_l1[:, :, None] * eye8[:, None, :]).reshape(64, 8)
    A_r = (attn_r1[:, :, None] * eye8[:, None, :]).reshape(64, 8)
    P = jnp.repeat(eye8, 8, axis=1)
    A2l = jnp.broadcast_to(attn_l2.reshape(40, 1), (40, 16)).astype(f32)
    A2r = jnp.broadcast_to(attn_r2.reshape(40, 1), (40, 16)).astype(f32)

    pad = EP - E
    tailpad = (TPAD - TCHUNK) * CHUNK
    srcp = jnp.concatenate([edge_index[0], jnp.zeros((pad + tailpad,),
                                                     jnp.int32)])
    dstp = jnp.concatenate([edge_index[1], jnp.full((pad,), N, jnp.int32),
                            jnp.zeros((tailpad,), jnp.int32)])
    srcp = srcp.reshape(TPAD, CHUNK)
    dstp = dstp.reshape(TPAD, CHUNK)

    table1, er1 = _tc_a(xp, W1, A_l, A_r)
    acc1 = _sc_edge(80, 64, True)(table1, er1, srcp, dstp)
    table2, er2 = _tc_b(acc1, W2, P, b1.reshape(1, 64), A2l, A2r)
    acc2 = _sc_edge(80, 64, True)(table2, er2, srcp, dstp)
    return _tc_c(acc2, b2.reshape(1, 40))
